# baseline (speedup 1.0000x reference)
import jax
import jax.numpy as jnp
from jax.experimental import pallas as pl

TOP_K = 8
N_EXPERTS = 72
BT = 2048


def _router_block(hs_ref, ow_ref, nw_ref, ob_ref, nb_ref,
                  logits_ref, tw_ref, ti_ref):
    x = hs_ref[0]
    w = jnp.concatenate([ow_ref[...], nw_ref[...]], axis=0)
    bias = jnp.concatenate([ob_ref[...], nb_ref[...]], axis=1)
    logits = jax.lax.dot_general(
        x, w, (((1,), (1,)), ((), ())),
        preferred_element_type=jnp.float32) + bias
    logits_ref[0] = logits

    lane_desc = (jnp.float32(127)
                 - jax.lax.broadcasted_iota(jnp.int32, (BT, N_EXPERTS), 1)
                 .astype(jnp.float32))
    neg = jnp.float32(-jnp.inf)
    cur = logits
    vals = []
    encs = []
    for _ in range(TOP_K):
        m = jnp.max(cur, axis=1, keepdims=True)
        enc = jnp.max(jnp.where(cur == m, lane_desc, neg), axis=1,
                      keepdims=True)
        vals.append(m)
        encs.append(enc)
        cur = jnp.where(lane_desc == enc, neg, cur)
    v = jnp.concatenate(vals, axis=1)
    e = jnp.concatenate(encs, axis=1)
    i = (jnp.float32(127) - e).astype(jnp.int32)
    sw = jax.nn.sigmoid(v)
    sw = sw / (jnp.sum(sw, axis=1, keepdims=True) + 1e-8)
    tw_ref[0] = sw
    ti_ref[0] = i


def kernel(hidden_states, orig_weight, orig_bias, new_weight, new_bias):
    b, s, d = hidden_states.shape
    ob = orig_bias.reshape(1, -1)
    nb = new_bias.reshape(1, -1)

    logits, tw, ti = pl.pallas_call(
        _router_block,
        grid=(b, s // BT),
        in_specs=[
            pl.BlockSpec((1, BT, d), lambda i, j: (i, j, 0)),
            pl.BlockSpec(orig_weight.shape, lambda i, j: (0, 0)),
            pl.BlockSpec(new_weight.shape, lambda i, j: (0, 0)),
            pl.BlockSpec(ob.shape, lambda i, j: (0, 0)),
            pl.BlockSpec(nb.shape, lambda i, j: (0, 0)),
        ],
        out_specs=[
            pl.BlockSpec((1, BT, N_EXPERTS), lambda i, j: (i, j, 0)),
            pl.BlockSpec((1, BT, TOP_K), lambda i, j: (i, j, 0)),
            pl.BlockSpec((1, BT, TOP_K), lambda i, j: (i, j, 0)),
        ],
        out_shape=[
            jax.ShapeDtypeStruct((b, s, N_EXPERTS), jnp.float32),
            jax.ShapeDtypeStruct((b, s, TOP_K), jnp.float32),
            jax.ShapeDtypeStruct((b, s, TOP_K), jnp.int32),
        ],
    )(hidden_states, orig_weight, new_weight, ob, nb)

    return (tw, ti, logits)
